# Initial kernel scaffold; baseline (speedup 1.0000x reference)
import jax
import jax.numpy as jnp
from jax.experimental import pallas as pl


def kernel(preds, target, batch_map):
    raise NotImplementedError("write your pallas kernel here")


# speedup vs baseline: 15.9429x; 15.9429x over previous
import functools

import jax
import jax.numpy as jnp
from jax import lax
from jax.experimental import pallas as pl
from jax.experimental.pallas import tpu as pltpu
from jax.experimental.pallas import tpu_sc as plsc

_N = 1048576
_S = 8192
_NC = 2
_NS = 16
_NW = _NC * _NS
_CHUNK = _N // _NW
_L = 16
_EPS = 1e-8


def _sc_partials(preds, target, batch_map):
  mesh = plsc.VectorSubcoreMesh(core_axis_name="c", subcore_axis_name="s")

  @functools.partial(
      pl.kernel,
      mesh=mesh,
      out_type=jax.ShapeDtypeStruct((3, _NW, _S), jnp.float32),
      compiler_params=pltpu.CompilerParams(
          use_tc_tiling_on_sc=False, needs_layout_passes=False),
      scratch_types=[
          pltpu.VMEM((_CHUNK,), jnp.float32),
          pltpu.VMEM((_CHUNK,), jnp.float32),
          pltpu.VMEM((_CHUNK,), jnp.int32),
          pltpu.VMEM((_S,), jnp.float32),
          pltpu.VMEM((_S,), jnp.float32),
          pltpu.VMEM((_S,), jnp.float32),
      ],
  )
  def sc_kernel(preds_hbm, target_hbm, ids_hbm, out_hbm,
                p_v, t_v, i_v, acc_p, acc_t, acc_d):
    wid = lax.axis_index("s") * _NC + lax.axis_index("c")
    base = wid * _CHUNK
    pltpu.sync_copy(preds_hbm.at[pl.ds(base, _CHUNK)], p_v)
    pltpu.sync_copy(target_hbm.at[pl.ds(base, _CHUNK)], t_v)
    pltpu.sync_copy(ids_hbm.at[pl.ds(base, _CHUNK)], i_v)

    zeros = jnp.zeros((_L,), jnp.float32)

    def zero_body(j, _):
      acc_p[pl.ds(j * _L, _L)] = zeros
      acc_t[pl.ds(j * _L, _L)] = zeros
      acc_d[pl.ds(j * _L, _L)] = zeros
      return _

    lax.fori_loop(0, _S // _L, zero_body, None)

    def body(i, _):
      off = i * _L
      ids = i_v[pl.ds(off, _L)]
      p = p_v[pl.ds(off, _L)]
      t = t_v[pl.ds(off, _L)]
      plsc.addupdate_scatter(acc_p, [ids], p * p)
      plsc.addupdate_scatter(acc_t, [ids], t * t)
      plsc.addupdate_scatter(acc_d, [ids], p * t)
      return _

    lax.fori_loop(0, _CHUNK // _L, body, None)

    pltpu.sync_copy(acc_p, out_hbm.at[0, wid])
    pltpu.sync_copy(acc_t, out_hbm.at[1, wid])
    pltpu.sync_copy(acc_d, out_hbm.at[2, wid])

  return sc_kernel(preds, target, batch_map)


def _tc_finish(sp_parts, st_parts, dot_parts):
  def tc_kernel(sp_ref, st_ref, dot_ref, out_ref):
    sp = jnp.sum(sp_ref[...], axis=0)
    st = jnp.sum(st_ref[...], axis=0)
    dot = jnp.sum(dot_ref[...], axis=0)
    pn = jnp.sqrt(sp) + _EPS
    tn = jnp.sqrt(st) + _EPS
    cos = jnp.clip(dot / (pn * tn), -1.0, 1.0)
    out_ref[0, 0] = 1.0 - jnp.sum(cos) / _S

  return pl.pallas_call(
      tc_kernel,
      out_shape=jax.ShapeDtypeStruct((1, 1), jnp.float32),
      out_specs=pl.BlockSpec(memory_space=pltpu.SMEM),
  )(sp_parts, st_parts, dot_parts)


def kernel(preds, target, batch_map):
  parts = _sc_partials(preds, target, batch_map)
  out = _tc_finish(parts[0], parts[1], parts[2])
  return out[0, 0]


# speedup vs baseline: 33.1668x; 2.0803x over previous
import functools

import jax
import jax.numpy as jnp
from jax import lax
from jax.experimental import pallas as pl
from jax.experimental.pallas import tpu as pltpu
from jax.experimental.pallas import tpu_sc as plsc

_N = 1048576
_S = 8192
_NC = 2
_NS = 16
_NW = _NC * _NS
_CHUNK = _N // _NW
_L = 16
_SPAD = _S + _L
_EPS = 1e-8


def _sc_partials(preds, target, batch_map):
  mesh = plsc.VectorSubcoreMesh(core_axis_name="c", subcore_axis_name="s")

  @functools.partial(
      pl.kernel,
      mesh=mesh,
      out_type=jax.ShapeDtypeStruct((3, _NW, _S), jnp.float32),
      compiler_params=pltpu.CompilerParams(
          use_tc_tiling_on_sc=False, needs_layout_passes=False),
      scratch_types=[
          pltpu.VMEM((_CHUNK,), jnp.float32),
          pltpu.VMEM((_CHUNK,), jnp.float32),
          pltpu.VMEM((_CHUNK + _L,), jnp.int32),
          pltpu.VMEM((_SPAD,), jnp.float32),
          pltpu.VMEM((_SPAD,), jnp.float32),
          pltpu.VMEM((_SPAD,), jnp.float32),
      ],
  )
  def sc_kernel(preds_hbm, target_hbm, ids_hbm, out_hbm,
                p_v, t_v, i_v, acc_p, acc_t, acc_d):
    wid = lax.axis_index("s") * _NC + lax.axis_index("c")
    base = wid * _CHUNK
    pltpu.sync_copy(preds_hbm.at[pl.ds(base, _CHUNK)], p_v)
    pltpu.sync_copy(target_hbm.at[pl.ds(base, _CHUNK)], t_v)
    pltpu.sync_copy(ids_hbm.at[pl.ds(base, _CHUNK)], i_v.at[pl.ds(0, _CHUNK)])
    i_v[pl.ds(_CHUNK, _L)] = jnp.full((_L,), _S, jnp.int32)

    zeros = jnp.zeros((_L,), jnp.float32)

    def zero_body(j, _):
      acc_p[pl.ds(j * _L, _L)] = zeros
      acc_t[pl.ds(j * _L, _L)] = zeros
      acc_d[pl.ds(j * _L, _L)] = zeros
      return _

    lax.fori_loop(0, _SPAD // _L, zero_body, None)

    def body(i, carries):
      cp, ct, cd = carries
      off = i * _L
      d = i_v[pl.ds(off, _L)]
      d2 = i_v[pl.ds(off + 1, _L)]
      m = d != d2
      p = p_v[pl.ds(off, _L)]
      t = t_v[pl.ds(off, _L)]

      def one(acc, v, carry):
        cl = plsc.cumsum(v)
        g = cl + carry
        plsc.addupdate_scatter(acc, [d], g, mask=m)
        plsc.addupdate_scatter(acc, [d2], -g, mask=m)
        return carry + cl[15]

      cp = one(acc_p, p * p, cp)
      ct = one(acc_t, t * t, ct)
      cd = one(acc_d, p * t, cd)
      return cp, ct, cd

    zero = jnp.float32(0)
    lax.fori_loop(0, _CHUNK // _L, body, (zero, zero, zero))

    pltpu.sync_copy(acc_p.at[pl.ds(0, _S)], out_hbm.at[0, wid])
    pltpu.sync_copy(acc_t.at[pl.ds(0, _S)], out_hbm.at[1, wid])
    pltpu.sync_copy(acc_d.at[pl.ds(0, _S)], out_hbm.at[2, wid])

  return sc_kernel(preds, target, batch_map)


def _tc_finish(sp_parts, st_parts, dot_parts):
  def tc_kernel(sp_ref, st_ref, dot_ref, out_ref):
    sp = jnp.sum(sp_ref[...], axis=0)
    st = jnp.sum(st_ref[...], axis=0)
    dot = jnp.sum(dot_ref[...], axis=0)
    pn = jnp.sqrt(sp) + _EPS
    tn = jnp.sqrt(st) + _EPS
    cos = jnp.clip(dot / (pn * tn), -1.0, 1.0)
    out_ref[0, 0] = 1.0 - jnp.sum(cos) / _S

  return pl.pallas_call(
      tc_kernel,
      out_shape=jax.ShapeDtypeStruct((1, 1), jnp.float32),
      out_specs=pl.BlockSpec(memory_space=pltpu.SMEM),
  )(sp_parts, st_parts, dot_parts)


def kernel(preds, target, batch_map):
  parts = _sc_partials(preds, target, batch_map)
  out = _tc_finish(parts[0], parts[1], parts[2])
  return out[0, 0]


# speedup vs baseline: 34.0237x; 1.0258x over previous
import functools

import jax
import jax.numpy as jnp
from jax import lax
from jax.experimental import pallas as pl
from jax.experimental.pallas import tpu as pltpu
from jax.experimental.pallas import tpu_sc as plsc

_N = 1048576
_S = 8192
_NC = 2
_NS = 16
_NW = _NC * _NS
_CHUNK = _N // _NW
_L = 16
_SPAD = _S + _L
_EPS = 1e-8

_TAKE_DNUMS = lax.GatherDimensionNumbers(
    offset_dims=(), collapsed_slice_dims=(0,), start_index_map=(0,))


def _bcast_last(x):
  idx = jnp.full((_L, 1), _L - 1, jnp.int32)
  return lax.gather(x, idx, _TAKE_DNUMS, (1,),
                    mode=lax.GatherScatterMode.PROMISE_IN_BOUNDS)


def _sc_partials(preds, target, batch_map):
  mesh = plsc.VectorSubcoreMesh(core_axis_name="c", subcore_axis_name="s")

  @functools.partial(
      pl.kernel,
      mesh=mesh,
      out_type=jax.ShapeDtypeStruct((3, _NW, _S), jnp.float32),
      compiler_params=pltpu.CompilerParams(
          use_tc_tiling_on_sc=False, needs_layout_passes=False),
      scratch_types=[
          pltpu.VMEM((_CHUNK,), jnp.float32),
          pltpu.VMEM((_CHUNK,), jnp.float32),
          pltpu.VMEM((_CHUNK + _L,), jnp.int32),
          pltpu.VMEM((_SPAD,), jnp.float32),
          pltpu.VMEM((_SPAD,), jnp.float32),
          pltpu.VMEM((_SPAD,), jnp.float32),
          pltpu.SemaphoreType.DMA,
      ],
  )
  def sc_kernel(preds_hbm, target_hbm, ids_hbm, out_hbm,
                p_v, t_v, i_v, acc_p, acc_t, acc_d, sem):
    wid = lax.axis_index("s") * _NC + lax.axis_index("c")
    base = wid * _CHUNK
    c1 = pltpu.async_copy(preds_hbm.at[pl.ds(base, _CHUNK)], p_v, sem)
    c2 = pltpu.async_copy(target_hbm.at[pl.ds(base, _CHUNK)], t_v, sem)
    c3 = pltpu.async_copy(
        ids_hbm.at[pl.ds(base, _CHUNK)], i_v.at[pl.ds(0, _CHUNK)], sem)
    i_v[pl.ds(_CHUNK, _L)] = jnp.full((_L,), _S, jnp.int32)

    zeros = jnp.zeros((_L,), jnp.float32)

    def zero_body(j, _):
      acc_p[pl.ds(j * _L, _L)] = zeros
      acc_t[pl.ds(j * _L, _L)] = zeros
      acc_d[pl.ds(j * _L, _L)] = zeros
      return _

    lax.fori_loop(0, _SPAD // _L, zero_body, None)
    c1.wait()
    c2.wait()
    c3.wait()

    def body(i, carries):
      cp, ct, cd = carries
      off = i * _L
      d = i_v[pl.ds(off, _L)]
      d2 = i_v[pl.ds(off + 1, _L)]
      m = d != d2
      p = p_v[pl.ds(off, _L)]
      t = t_v[pl.ds(off, _L)]

      def one(acc, v, carry):
        g = plsc.cumsum(v) + carry
        plsc.addupdate_scatter(acc, [d], g, mask=m)
        plsc.addupdate_scatter(acc, [d2], -g, mask=m)
        return _bcast_last(g)

      cp = one(acc_p, p * p, cp)
      ct = one(acc_t, t * t, ct)
      cd = one(acc_d, p * t, cd)
      return cp, ct, cd

    lax.fori_loop(0, _CHUNK // _L, body, (zeros, zeros, zeros))

    pltpu.sync_copy(acc_p.at[pl.ds(0, _S)], out_hbm.at[0, wid])
    pltpu.sync_copy(acc_t.at[pl.ds(0, _S)], out_hbm.at[1, wid])
    pltpu.sync_copy(acc_d.at[pl.ds(0, _S)], out_hbm.at[2, wid])

  return sc_kernel(preds, target, batch_map)


def _tc_finish(parts):
  def tc_kernel(parts_ref, out_ref):
    sp = jnp.sum(parts_ref[0], axis=0)
    st = jnp.sum(parts_ref[1], axis=0)
    dot = jnp.sum(parts_ref[2], axis=0)
    pn = jnp.sqrt(sp) + _EPS
    tn = jnp.sqrt(st) + _EPS
    cos = jnp.clip(dot / (pn * tn), -1.0, 1.0)
    out_ref[0, 0] = 1.0 - jnp.sum(cos) / _S

  return pl.pallas_call(
      tc_kernel,
      out_shape=jax.ShapeDtypeStruct((1, 1), jnp.float32),
      out_specs=pl.BlockSpec(memory_space=pltpu.SMEM),
  )(parts)


def kernel(preds, target, batch_map):
  return _tc_finish(_sc_partials(preds, target, batch_map))[0, 0]


# speedup vs baseline: 35.9224x; 1.0558x over previous
import functools

import jax
import jax.numpy as jnp
from jax import lax
from jax.experimental import pallas as pl
from jax.experimental.pallas import tpu as pltpu
from jax.experimental.pallas import tpu_sc as plsc

_N = 1048576
_S = 8192
_NC = 2
_NS = 16
_NW = _NC * _NS
_CHUNK = _N // _NW
_L = 16
_SPAD = _S + _L
_EPS = 1e-8

_TAKE_DNUMS = lax.GatherDimensionNumbers(
    offset_dims=(), collapsed_slice_dims=(0,), start_index_map=(0,))


def _bcast_last(x):
  idx = jnp.full((_L, 1), _L - 1, jnp.int32)
  return lax.gather(x, idx, _TAKE_DNUMS, (1,),
                    mode=lax.GatherScatterMode.PROMISE_IN_BOUNDS)


def _sc_partials(preds, target, batch_map):
  mesh = plsc.VectorSubcoreMesh(core_axis_name="c", subcore_axis_name="s")

  @functools.partial(
      pl.kernel,
      mesh=mesh,
      out_type=jax.ShapeDtypeStruct((3, _NW, _S), jnp.float32),
      compiler_params=pltpu.CompilerParams(
          use_tc_tiling_on_sc=False, needs_layout_passes=False),
      scratch_types=[
          pltpu.VMEM((_CHUNK,), jnp.float32),
          pltpu.VMEM((_CHUNK,), jnp.float32),
          pltpu.VMEM((_CHUNK + _L,), jnp.int32),
          pltpu.VMEM((_SPAD,), jnp.float32),
          pltpu.VMEM((_SPAD,), jnp.float32),
          pltpu.VMEM((_SPAD,), jnp.float32),
          pltpu.SemaphoreType.DMA,
      ],
  )
  def sc_kernel(preds_hbm, target_hbm, ids_hbm, out_hbm,
                p_v, t_v, i_v, acc_p, acc_t, acc_d, sem):
    wid = lax.axis_index("s") * _NC + lax.axis_index("c")
    base = wid * _CHUNK
    c1 = pltpu.async_copy(preds_hbm.at[pl.ds(base, _CHUNK)], p_v, sem)
    c2 = pltpu.async_copy(target_hbm.at[pl.ds(base, _CHUNK)], t_v, sem)
    c3 = pltpu.async_copy(
        ids_hbm.at[pl.ds(base, _CHUNK)], i_v.at[pl.ds(0, _CHUNK)], sem)
    i_v[pl.ds(_CHUNK, _L)] = jnp.full((_L,), _S, jnp.int32)

    zeros = jnp.zeros((_L,), jnp.float32)

    def zero_body(j, _):
      acc_p[pl.ds(j * _L, _L)] = zeros
      acc_t[pl.ds(j * _L, _L)] = zeros
      acc_d[pl.ds(j * _L, _L)] = zeros
      return _

    lax.fori_loop(0, _SPAD // _L, zero_body, None)
    c1.wait()
    c2.wait()
    c3.wait()

    def body(i, carries):
      cp, ct, cd = carries
      off = i * _L
      d = i_v[pl.ds(off, _L)]
      d2 = i_v[pl.ds(off + 1, _L)]
      m = d != d2
      p = p_v[pl.ds(off, _L)]
      t = t_v[pl.ds(off, _L)]

      def one(acc, v, carry):
        cl = plsc.cumsum(v)
        g = cl + carry
        plsc.addupdate_scatter(acc, [d], g, mask=m)
        plsc.addupdate_scatter(acc, [d2], -g, mask=m)
        return carry + _bcast_last(cl)

      cp = one(acc_p, p * p, cp)
      ct = one(acc_t, t * t, ct)
      cd = one(acc_d, p * t, cd)
      return cp, ct, cd

    lax.fori_loop(0, _CHUNK // _L, body, (zeros, zeros, zeros), unroll=4)

    pltpu.sync_copy(acc_p.at[pl.ds(0, _S)], out_hbm.at[0, wid])
    pltpu.sync_copy(acc_t.at[pl.ds(0, _S)], out_hbm.at[1, wid])
    pltpu.sync_copy(acc_d.at[pl.ds(0, _S)], out_hbm.at[2, wid])

  return sc_kernel(preds, target, batch_map)


def _tc_finish(parts):
  def tc_kernel(parts_ref, out_ref):
    sp = jnp.sum(parts_ref[0], axis=0)
    st = jnp.sum(parts_ref[1], axis=0)
    dot = jnp.sum(parts_ref[2], axis=0)
    pn = jnp.sqrt(sp) + _EPS
    tn = jnp.sqrt(st) + _EPS
    cos = jnp.clip(dot / (pn * tn), -1.0, 1.0)
    out_ref[0, 0] = 1.0 - jnp.sum(cos) / _S

  return pl.pallas_call(
      tc_kernel,
      out_shape=jax.ShapeDtypeStruct((1, 1), jnp.float32),
      out_specs=pl.BlockSpec(memory_space=pltpu.SMEM),
  )(parts)


def kernel(preds, target, batch_map):
  return _tc_finish(_sc_partials(preds, target, batch_map))[0, 0]


# speedup vs baseline: 37.9808x; 1.0573x over previous
import functools

import jax
import jax.numpy as jnp
from jax import lax
from jax.experimental import pallas as pl
from jax.experimental.pallas import tpu as pltpu
from jax.experimental.pallas import tpu_sc as plsc

_N = 1048576
_S = 8192
_NC = 2
_NS = 16
_NW = _NC * _NS
_CHUNK = _N // _NW
_L = 16
_EPS = 1e-8


def _sc_partials(preds, target, batch_map):
  mesh = plsc.VectorSubcoreMesh(core_axis_name="c", subcore_axis_name="s")

  @functools.partial(
      pl.kernel,
      mesh=mesh,
      out_type=jax.ShapeDtypeStruct((3, _NW, _S), jnp.float32),
      compiler_params=pltpu.CompilerParams(
          use_tc_tiling_on_sc=False, needs_layout_passes=False),
      scratch_types=[
          pltpu.VMEM((_CHUNK,), jnp.float32),
          pltpu.VMEM((_CHUNK,), jnp.float32),
          pltpu.VMEM((_CHUNK + _L,), jnp.int32),
          pltpu.VMEM((_S,), jnp.float32),
          pltpu.VMEM((_S,), jnp.float32),
          pltpu.VMEM((_S,), jnp.float32),
          pltpu.SemaphoreType.DMA,
      ],
  )
  def sc_kernel(preds_hbm, target_hbm, ids_hbm, out_hbm,
                p_v, t_v, i_v, acc_p, acc_t, acc_d, sem):
    wid = lax.axis_index("s") * _NC + lax.axis_index("c")
    base = wid * _CHUNK
    c1 = pltpu.async_copy(preds_hbm.at[pl.ds(base, _CHUNK)], p_v, sem)
    c2 = pltpu.async_copy(target_hbm.at[pl.ds(base, _CHUNK)], t_v, sem)
    c3 = pltpu.async_copy(
        ids_hbm.at[pl.ds(base, _CHUNK)], i_v.at[pl.ds(0, _CHUNK)], sem)
    i_v[pl.ds(_CHUNK, _L)] = jnp.zeros((_L,), jnp.int32)

    zeros = jnp.zeros((_L,), jnp.float32)

    def zero_body(j, _):
      acc_p[pl.ds(j * _L, _L)] = zeros
      acc_t[pl.ds(j * _L, _L)] = zeros
      acc_d[pl.ds(j * _L, _L)] = zeros
      return _

    lax.fori_loop(0, _S // _L, zero_body, None)
    c1.wait()
    c2.wait()
    c3.wait()

    id0 = i_v[pl.ds(0, _L)]

    def body(j, state):
      idc, runp, runt, rund = state
      off = j * _L
      pv = p_v[pl.ds(off, _L)]
      tv = t_v[pl.ds(off, _L)]
      idn = i_v[pl.ds(off + _L, _L)]
      m = idc != idn

      def one(acc, run, prod):
        nr = run + prod
        plsc.addupdate_scatter(acc, [idc], nr, mask=m)
        return jnp.where(m, 0.0, nr)

      runp = one(acc_p, runp, pv * pv)
      runt = one(acc_t, runt, tv * tv)
      rund = one(acc_d, rund, pv * tv)
      return idn, runp, runt, rund

    idc, runp, runt, rund = lax.fori_loop(
        0, _CHUNK // _L, body, (id0, zeros, zeros, zeros), unroll=4)
    plsc.addupdate_scatter(acc_p, [idc], runp)
    plsc.addupdate_scatter(acc_t, [idc], runt)
    plsc.addupdate_scatter(acc_d, [idc], rund)

    pltpu.sync_copy(acc_p, out_hbm.at[0, wid])
    pltpu.sync_copy(acc_t, out_hbm.at[1, wid])
    pltpu.sync_copy(acc_d, out_hbm.at[2, wid])

  return sc_kernel(preds, target, batch_map)


def _tc_finish(parts):
  def tc_kernel(parts_ref, out_ref):
    sp = jnp.sum(parts_ref[0], axis=0)
    st = jnp.sum(parts_ref[1], axis=0)
    dot = jnp.sum(parts_ref[2], axis=0)
    pn = jnp.sqrt(sp) + _EPS
    tn = jnp.sqrt(st) + _EPS
    cos = jnp.clip(dot / (pn * tn), -1.0, 1.0)
    out_ref[0, 0] = 1.0 - jnp.sum(cos) / _S

  return pl.pallas_call(
      tc_kernel,
      out_shape=jax.ShapeDtypeStruct((1, 1), jnp.float32),
      out_specs=pl.BlockSpec(memory_space=pltpu.SMEM),
  )(parts)


def kernel(preds, target, batch_map):
  return _tc_finish(_sc_partials(preds, target, batch_map))[0, 0]
